# baseline (speedup 1.0000x reference)
import jax
import jax.numpy as jnp
from jax.experimental import pallas as pl
from jax.experimental.pallas import tpu as pltpu


def _add_kernel(x_ref, pe_ref, o_ref):
    o_ref[...] = x_ref[...] + pe_ref[...]


def kernel(x, pe):
    B, S, D = x.shape
    S_BLK = 2048
    D_BLK = 512
    return pl.pallas_call(
        _add_kernel,
        grid=(S // S_BLK, D // D_BLK, B),
        in_specs=[
            pl.BlockSpec((1, S_BLK, D_BLK), lambda i, k, j: (j, i, k)),
            pl.BlockSpec((S_BLK, D_BLK), lambda i, k, j: (i, k)),
        ],
        out_specs=pl.BlockSpec((1, S_BLK, D_BLK), lambda i, k, j: (j, i, k)),
        out_shape=jax.ShapeDtypeStruct(x.shape, x.dtype),
        compiler_params=pltpu.CompilerParams(vmem_limit_bytes=60 * 1024 * 1024),
    )(x, pe)


# speedup vs baseline: 1.0713x; 1.0713x over previous
import jax
import jax.numpy as jnp
from jax.experimental import pallas as pl
from jax.experimental.pallas import tpu as pltpu


def _add_kernel(x_ref, pe_ref, o_ref):
    o_ref[...] = x_ref[...] + pe_ref[...]


def kernel(x, pe):
    B, S, D = x.shape
    S_BLK = 2048
    return pl.pallas_call(
        _add_kernel,
        grid=(S // S_BLK, B),
        in_specs=[
            pl.BlockSpec((1, S_BLK, D), lambda i, j: (j, i, 0)),
            pl.BlockSpec((S_BLK, D), lambda i, j: (i, 0)),
        ],
        out_specs=pl.BlockSpec((1, S_BLK, D), lambda i, j: (j, i, 0)),
        out_shape=jax.ShapeDtypeStruct(x.shape, x.dtype),
        compiler_params=pltpu.CompilerParams(vmem_limit_bytes=60 * 1024 * 1024),
    )(x, pe)
